# Initial kernel scaffold; baseline (speedup 1.0000x reference)
import jax
import jax.numpy as jnp
from jax.experimental import pallas as pl


def kernel(x, edge_index, W1, b1, W3, b3):
    raise NotImplementedError("write your pallas kernel here")


# speedup vs baseline: 11.2711x; 11.2711x over previous
import functools

import jax
import jax.numpy as jnp
from jax import lax
from jax.experimental import pallas as pl
from jax.experimental.pallas import tpu as pltpu
from jax.experimental.pallas import tpu_sc as plsc

K_ITERS = 10
ALPHA = 0.1

NC = 2
NS = 16
NW = NC * NS
CH = 128


def _mesh():
    return plsc.VectorSubcoreMesh(
        core_axis_name="c", subcore_axis_name="s", num_cores=NC, num_subcores=NS
    )


def _make_prop(NP, F, NCH):
    RPT = NP // NS

    @functools.partial(
        pl.kernel,
        out_type=jax.ShapeDtypeStruct((NC, NP, F), jnp.float32),
        mesh=_mesh(),
        scratch_types=[
            pltpu.VMEM((NCH, CH), jnp.int32),
            pltpu.VMEM((NCH, CH), jnp.int32),
            pltpu.VMEM((CH, F), jnp.float32),
            pltpu.VMEM_SHARED((NP, F), jnp.float32),
            pltpu.SemaphoreType.DMA,
        ],
        compiler_params=pltpu.CompilerParams(use_tc_tiling_on_sc=False),
    )
    def prop(g_hbm, srcp_hbm, dstp_hbm, zeros_hbm, y_hbm, src_v, dst_v, buf, acc_sh, sem):
        c = lax.axis_index("c")
        s = lax.axis_index("s")
        wid = s * NC + c
        pltpu.sync_copy(srcp_hbm.at[wid], src_v)
        pltpu.sync_copy(dstp_hbm.at[wid], dst_v)
        pltpu.sync_copy(zeros_hbm.at[pl.ds(s * RPT, RPT)], acc_sh.at[pl.ds(s * RPT, RPT)])
        plsc.subcore_barrier()

        def body(j, carry):
            pltpu.async_copy(g_hbm.at[src_v.at[j]], buf, sem).wait()
            pltpu.sync_copy(buf, acc_sh.at[dst_v.at[j]], add=True)
            return carry

        lax.fori_loop(0, NCH, body, 0)
        plsc.subcore_barrier()
        pltpu.sync_copy(acc_sh.at[pl.ds(s * RPT, RPT)], y_hbm.at[c, pl.ds(s * RPT, RPT)])

    return prop


def _make_deg(NP, NCH):
    F = 16
    RPT = NP // NS

    @functools.partial(
        pl.kernel,
        out_type=jax.ShapeDtypeStruct((NC, NP, F), jnp.float32),
        mesh=_mesh(),
        scratch_types=[
            pltpu.VMEM((NCH, CH), jnp.int32),
            pltpu.VMEM((CH, F), jnp.float32),
            pltpu.VMEM_SHARED((NP, F), jnp.float32),
        ],
        compiler_params=pltpu.CompilerParams(use_tc_tiling_on_sc=False),
    )
    def deg(dstp_hbm, ones_hbm, zeros_hbm, y_hbm, dst_v, buf, acc_sh):
        c = lax.axis_index("c")
        s = lax.axis_index("s")
        wid = s * NC + c
        pltpu.sync_copy(dstp_hbm.at[wid], dst_v)
        pltpu.sync_copy(ones_hbm, buf)
        pltpu.sync_copy(zeros_hbm.at[pl.ds(s * RPT, RPT)], acc_sh.at[pl.ds(s * RPT, RPT)])
        plsc.subcore_barrier()

        def body(j, carry):
            pltpu.sync_copy(buf, acc_sh.at[dst_v.at[j]], add=True)
            return carry

        lax.fori_loop(0, NCH, body, 0)
        plsc.subcore_barrier()
        pltpu.sync_copy(acc_sh.at[pl.ds(s * RPT, RPT)], y_hbm.at[c, pl.ds(s * RPT, RPT)])

    return deg


def _tc_call(body, out_shapes, *args):
    return pl.pallas_call(
        body,
        out_shape=out_shapes,
        compiler_params=pltpu.CompilerParams(vmem_limit_bytes=100 * 1024 * 1024),
    )(*args)


def _stats_and_a(y_deg, x_pad, W1):
    def body(yd_ref, x_ref, w_ref, a_ref, dinv_ref, cc_ref, sqm_ref):
        deg = yd_ref[0, :, 0:1] + yd_ref[1, :, 0:1]
        m = jnp.maximum(deg + 1.0, 1.0)
        dinv = lax.rsqrt(m)
        dinv_ref[...] = dinv
        cc_ref[...] = dinv * dinv
        sqm_ref[...] = jnp.sqrt(m)
        xw = jnp.dot(
            x_ref[...], w_ref[...],
            preferred_element_type=jnp.float32,
            precision=lax.Precision.HIGHEST,
        )
        a_ref[...] = dinv * xw

    NP = x_pad.shape[0]
    H = W1.shape[1]
    return _tc_call(
        body,
        (
            jax.ShapeDtypeStruct((NP, H), jnp.float32),
            jax.ShapeDtypeStruct((NP, 1), jnp.float32),
            jax.ShapeDtypeStruct((NP, 1), jnp.float32),
            jax.ShapeDtypeStruct((NP, 1), jnp.float32),
        ),
        y_deg, x_pad, W1,
    )


def _mk_u0(ya, a, dinv, b1):
    def body(ya_ref, a_ref, dinv_ref, b1_ref, u0_ref):
        s = ya_ref[0] + ya_ref[1] + a_ref[...]
        dinv = dinv_ref[...]
        u0_ref[...] = dinv * (dinv * s + b1_ref[...])

    return _tc_call(
        body, jax.ShapeDtypeStruct(a.shape, jnp.float32),
        ya, a, dinv, b1.reshape(1, -1),
    )


def _combine(y, u, cc, u0):
    def body(y_ref, u_ref, cc_ref, u0_ref, out_ref):
        s = y_ref[0] + y_ref[1] + u_ref[...]
        out_ref[...] = (1.0 - ALPHA) * cc_ref[...] * s + ALPHA * u0_ref[...]

    return _tc_call(body, jax.ShapeDtypeStruct(u.shape, jnp.float32), y, u, cc, u0)


def _mm_v(u, sqm, dinv, W3):
    def body(u_ref, sqm_ref, dinv_ref, w_ref, v_ref):
        z = sqm_ref[...] * u_ref[...]
        v_ref[...] = dinv_ref[...] * jnp.dot(
            z, w_ref[...],
            preferred_element_type=jnp.float32,
            precision=lax.Precision.HIGHEST,
        )

    NP = u.shape[0]
    C = W3.shape[1]
    return _tc_call(body, jax.ShapeDtypeStruct((NP, C), jnp.float32), u, sqm, dinv, W3)


def _final(yv, v, dinv, b3):
    def body(yv_ref, v_ref, dinv_ref, b3_ref, out_ref):
        s = yv_ref[0] + yv_ref[1] + v_ref[...]
        out_ref[...] = dinv_ref[...] * s + b3_ref[...]

    return _tc_call(
        body, jax.ShapeDtypeStruct(v.shape, jnp.float32), yv, v, dinv, b3.reshape(1, -1)
    )


def kernel(x, edge_index, W1, b1, W3, b3):
    n = x.shape[0]
    e = edge_index.shape[1]
    in_feats = x.shape[1]
    h_feats = W1.shape[1]
    num_classes = W3.shape[1]

    NP = ((n + 1 + NS * 8 - 1) // (NS * 8)) * (NS * 8)
    NCH = -(-e // (NW * CH))
    e_pad = NW * NCH * CH

    src = jnp.concatenate([edge_index[0], jnp.full((e_pad - e,), n, jnp.int32)])
    dst = jnp.concatenate([edge_index[1], jnp.full((e_pad - e,), n, jnp.int32)])
    srcp = src.reshape(NW, NCH, CH)
    dstp = dst.reshape(NW, NCH, CH)

    x_pad = jnp.zeros((NP, in_feats), jnp.float32).at[:n].set(x)
    zeros_h = jnp.zeros((NP, h_feats), jnp.float32)
    zeros_16 = jnp.zeros((NP, 16), jnp.float32)
    ones_buf = jnp.ones((CH, 16), jnp.float32)

    prop_h = _make_prop(NP, h_feats, NCH)
    prop_c = _make_prop(NP, 16, NCH)
    deg_k = _make_deg(NP, NCH)

    y_deg = deg_k(dstp, ones_buf, zeros_16)
    a, dinv, cc, sqm = _stats_and_a(y_deg, x_pad, W1)

    ya = prop_h(a, srcp, dstp, zeros_h)
    u0 = _mk_u0(ya, a, dinv, b1)

    u = u0
    for _ in range(K_ITERS):
        y = prop_h(u, srcp, dstp, zeros_h)
        u = _combine(y, u, cc, u0)

    v_pad = _mm_v(u, sqm, dinv, W3)
    v16 = jnp.zeros((NP, 16), jnp.float32).at[:, :num_classes].set(v_pad)
    yv = prop_c(v16, srcp, dstp, zeros_16)
    out_full = _final(yv, v16, dinv, b3)
    return out_full[:n, :num_classes]


# speedup vs baseline: 14.6139x; 1.2966x over previous
import functools

import jax
import jax.numpy as jnp
from jax import lax
from jax.experimental import pallas as pl
from jax.experimental.pallas import tpu as pltpu
from jax.experimental.pallas import tpu_sc as plsc

K_ITERS = 10
ALPHA = 0.1

NC = 2
NS = 16
NW = NC * NS
CH = 128
NBUF = 4


def _mesh():
    return plsc.VectorSubcoreMesh(
        core_axis_name="c", subcore_axis_name="s", num_cores=NC, num_subcores=NS
    )


def _make_prop(NP, F, NCH):
    RPT = NP // NS

    @functools.partial(
        pl.kernel,
        out_type=jax.ShapeDtypeStruct((NC, NP, F), jnp.float32),
        mesh=_mesh(),
        scratch_types=[
            pltpu.VMEM((NCH, CH), jnp.int32),
            pltpu.VMEM((NCH, CH), jnp.int32),
            [pltpu.VMEM((CH, F), jnp.float32)] * NBUF,
            pltpu.VMEM_SHARED((NP, F), jnp.float32),
            [pltpu.SemaphoreType.DMA] * NBUF,
            [pltpu.SemaphoreType.DMA] * NBUF,
        ],
        compiler_params=pltpu.CompilerParams(use_tc_tiling_on_sc=False),
    )
    def prop(g_hbm, srcp_hbm, dstp_hbm, zeros_hbm, y_hbm, src_v, dst_v, bufs, acc_sh, gsems, ssems):
        c = lax.axis_index("c")
        s = lax.axis_index("s")
        wid = s * NC + c
        pltpu.sync_copy(srcp_hbm.at[wid], src_v)
        pltpu.sync_copy(dstp_hbm.at[wid], dst_v)
        pltpu.sync_copy(zeros_hbm.at[pl.ds(s * RPT, RPT)], acc_sh.at[pl.ds(s * RPT, RPT)])
        plsc.subcore_barrier()

        for b in range(NBUF - 1):
            pltpu.async_copy(g_hbm.at[src_v.at[b]], bufs[b], gsems[b])

        def body(j, carry):
            jn = j + NBUF - 1

            @pl.when(jn < NCH)
            def _():
                bn = lax.rem(jn, NBUF)
                for bb in range(NBUF):
                    @pl.when(bn == bb)
                    def _():
                        pltpu.async_copy(g_hbm.at[src_v.at[jn]], bufs[bb], gsems[bb])

            b = lax.rem(j, NBUF)
            for bb in range(NBUF):
                @pl.when(b == bb)
                def _():
                    pltpu.make_async_copy(g_hbm.at[src_v.at[j]], bufs[bb], gsems[bb]).wait()
                    pltpu.sync_copy(bufs[bb], acc_sh.at[dst_v.at[j]], add=True)
            return carry

        lax.fori_loop(0, NCH, body, 0)
        plsc.subcore_barrier()
        pltpu.sync_copy(acc_sh.at[pl.ds(s * RPT, RPT)], y_hbm.at[c, pl.ds(s * RPT, RPT)])

    return prop


def _make_deg(NP, NCH):
    F = 16
    RPT = NP // NS

    @functools.partial(
        pl.kernel,
        out_type=jax.ShapeDtypeStruct((NC, NP, F), jnp.float32),
        mesh=_mesh(),
        scratch_types=[
            pltpu.VMEM((NCH, CH), jnp.int32),
            pltpu.VMEM((CH, F), jnp.float32),
            pltpu.VMEM_SHARED((NP, F), jnp.float32),
        ],
        compiler_params=pltpu.CompilerParams(use_tc_tiling_on_sc=False),
    )
    def deg(dstp_hbm, ones_hbm, zeros_hbm, y_hbm, dst_v, buf, acc_sh):
        c = lax.axis_index("c")
        s = lax.axis_index("s")
        wid = s * NC + c
        pltpu.sync_copy(dstp_hbm.at[wid], dst_v)
        pltpu.sync_copy(ones_hbm, buf)
        pltpu.sync_copy(zeros_hbm.at[pl.ds(s * RPT, RPT)], acc_sh.at[pl.ds(s * RPT, RPT)])
        plsc.subcore_barrier()

        def body(j, carry):
            pltpu.sync_copy(buf, acc_sh.at[dst_v.at[j]], add=True)
            return carry

        lax.fori_loop(0, NCH, body, 0)
        plsc.subcore_barrier()
        pltpu.sync_copy(acc_sh.at[pl.ds(s * RPT, RPT)], y_hbm.at[c, pl.ds(s * RPT, RPT)])

    return deg


def _tc_call(body, out_shapes, *args):
    return pl.pallas_call(
        body,
        out_shape=out_shapes,
        compiler_params=pltpu.CompilerParams(vmem_limit_bytes=100 * 1024 * 1024),
    )(*args)


def _stats_and_a(y_deg, x_pad, W1):
    def body(yd_ref, x_ref, w_ref, a_ref, dinv_ref, cc_ref, sqm_ref):
        deg = yd_ref[0, :, 0:1] + yd_ref[1, :, 0:1]
        m = jnp.maximum(deg + 1.0, 1.0)
        dinv = lax.rsqrt(m)
        dinv_ref[...] = dinv
        cc_ref[...] = dinv * dinv
        sqm_ref[...] = jnp.sqrt(m)
        xw = jnp.dot(
            x_ref[...], w_ref[...],
            preferred_element_type=jnp.float32,
            precision=lax.Precision.HIGHEST,
        )
        a_ref[...] = dinv * xw

    NP = x_pad.shape[0]
    H = W1.shape[1]
    return _tc_call(
        body,
        (
            jax.ShapeDtypeStruct((NP, H), jnp.float32),
            jax.ShapeDtypeStruct((NP, 1), jnp.float32),
            jax.ShapeDtypeStruct((NP, 1), jnp.float32),
            jax.ShapeDtypeStruct((NP, 1), jnp.float32),
        ),
        y_deg, x_pad, W1,
    )


def _mk_u0(ya, a, dinv, b1):
    def body(ya_ref, a_ref, dinv_ref, b1_ref, u0_ref):
        s = ya_ref[0] + ya_ref[1] + a_ref[...]
        dinv = dinv_ref[...]
        u0_ref[...] = dinv * (dinv * s + b1_ref[...])

    return _tc_call(
        body, jax.ShapeDtypeStruct(a.shape, jnp.float32),
        ya, a, dinv, b1.reshape(1, -1),
    )


def _combine(y, u, cc, u0):
    def body(y_ref, u_ref, cc_ref, u0_ref, out_ref):
        s = y_ref[0] + y_ref[1] + u_ref[...]
        out_ref[...] = (1.0 - ALPHA) * cc_ref[...] * s + ALPHA * u0_ref[...]

    return _tc_call(body, jax.ShapeDtypeStruct(u.shape, jnp.float32), y, u, cc, u0)


def _mm_v(u, sqm, dinv, W3):
    def body(u_ref, sqm_ref, dinv_ref, w_ref, v_ref):
        z = sqm_ref[...] * u_ref[...]
        v_ref[...] = dinv_ref[...] * jnp.dot(
            z, w_ref[...],
            preferred_element_type=jnp.float32,
            precision=lax.Precision.HIGHEST,
        )

    NP = u.shape[0]
    C = W3.shape[1]
    return _tc_call(body, jax.ShapeDtypeStruct((NP, C), jnp.float32), u, sqm, dinv, W3)


def _final(yv, v, dinv, b3):
    def body(yv_ref, v_ref, dinv_ref, b3_ref, out_ref):
        s = yv_ref[0] + yv_ref[1] + v_ref[...]
        out_ref[...] = dinv_ref[...] * s + b3_ref[...]

    return _tc_call(
        body, jax.ShapeDtypeStruct(v.shape, jnp.float32), yv, v, dinv, b3.reshape(1, -1)
    )


def kernel(x, edge_index, W1, b1, W3, b3):
    n = x.shape[0]
    e = edge_index.shape[1]
    in_feats = x.shape[1]
    h_feats = W1.shape[1]
    num_classes = W3.shape[1]

    NP = ((n + 1 + NS * 8 - 1) // (NS * 8)) * (NS * 8)
    NCH = -(-e // (NW * CH))
    e_pad = NW * NCH * CH

    src = jnp.concatenate([edge_index[0], jnp.full((e_pad - e,), n, jnp.int32)])
    dst = jnp.concatenate([edge_index[1], jnp.full((e_pad - e,), n, jnp.int32)])
    srcp = src.reshape(NW, NCH, CH)
    dstp = dst.reshape(NW, NCH, CH)

    x_pad = jnp.zeros((NP, in_feats), jnp.float32).at[:n].set(x)
    zeros_h = jnp.zeros((NP, h_feats), jnp.float32)
    zeros_16 = jnp.zeros((NP, 16), jnp.float32)
    ones_buf = jnp.ones((CH, 16), jnp.float32)

    prop_h = _make_prop(NP, h_feats, NCH)
    prop_c = _make_prop(NP, 16, NCH)
    deg_k = _make_deg(NP, NCH)

    y_deg = deg_k(dstp, ones_buf, zeros_16)
    a, dinv, cc, sqm = _stats_and_a(y_deg, x_pad, W1)

    ya = prop_h(a, srcp, dstp, zeros_h)
    u0 = _mk_u0(ya, a, dinv, b1)

    u = u0
    for _ in range(K_ITERS):
        y = prop_h(u, srcp, dstp, zeros_h)
        u = _combine(y, u, cc, u0)

    v_pad = _mm_v(u, sqm, dinv, W3)
    v16 = jnp.zeros((NP, 16), jnp.float32).at[:, :num_classes].set(v_pad)
    yv = prop_c(v16, srcp, dstp, zeros_16)
    out_full = _final(yv, v16, dinv, b3)
    return out_full[:n, :num_classes]
